# baseline (speedup 1.0000x reference)
import functools

import jax
import jax.numpy as jnp
from jax import lax
from jax.experimental import pallas as pl
from jax.experimental.pallas import tpu as pltpu
from jax.experimental.pallas import tpu_sc as plsc

BATCH = 16384
SEQ = 200
EMBED = 32
NUM_WORKERS = 32
BPW = BATCH // NUM_WORKERS


def _make_kernel():
    mesh = plsc.VectorSubcoreMesh(core_axis_name="c", subcore_axis_name="s")

    @functools.partial(
        pl.kernel,
        mesh=mesh,
        out_type=jax.ShapeDtypeStruct((SEQ, EMBED, BATCH), jnp.float32),
        scratch_types=[
            pltpu.VMEM((2, BPW), jnp.int32),
            pltpu.VMEM((2, BPW, EMBED), jnp.float32),
            pltpu.VMEM((2, EMBED, BPW), jnp.float32),
            [pltpu.SemaphoreType.DMA] * 2,
            [pltpu.SemaphoreType.DMA] * 2,
            [pltpu.SemaphoreType.DMA] * 2,
        ],
        compiler_params=pltpu.CompilerParams(use_tc_tiling_on_sc=False,
                                             needs_layout_passes=False),
    )
    def gather_kernel(idx_hbm, table_hbm, out_hbm, idx_v, rows_v, tr_v,
                      idx_sems, out_sems, gsems):
        wid = lax.axis_index("s") * 2 + lax.axis_index("c")
        b0 = pl.multiple_of(wid * BPW, BPW)
        lane = lax.iota(jnp.int32, 16)
        evecs = [jnp.full((16,), e, jnp.int32) for e in range(EMBED)]

        def idx_copy(s, b):
            return pltpu.make_async_copy(
                idx_hbm.at[s, pl.ds(b0, BPW)], idx_v.at[b], idx_sems[b])

        def gather(b):
            return pltpu.make_async_copy(
                table_hbm.at[idx_v.at[b]], rows_v.at[b], gsems[b])

        def out_copy(s, b):
            return pltpu.make_async_copy(
                tr_v.at[b], out_hbm.at[s, :, pl.ds(b0, BPW)], out_sems[b])

        def transpose_block(b):
            rows = rows_v.at[b]
            tr = tr_v.at[b]

            def tbody(t, carry):
                r16 = t * 16
                rvec = lane + r16
                for e in range(EMBED):
                    v = plsc.load_gather(rows, [rvec, evecs[e]])
                    tr[e, pl.ds(r16, 16)] = v
                return carry

            lax.fori_loop(0, BPW // 16, tbody, 0)

        idx_copy(0, 0).start()
        idx_copy(0, 0).wait()
        gather(0).start()
        idx_copy(1, 1).start()

        def body(j, carry):
            for b in range(2):
                s = j * 2 + b
                bn = 1 - b
                @pl.when(s + 1 < SEQ)
                def _():
                    idx_copy(s + 1, bn).wait()
                    gather(bn).start()
                gather(b).wait()
                @pl.when(s + 2 < SEQ)
                def _():
                    idx_copy(s + 2, b).start()
                @pl.when(s >= 2)
                def _():
                    out_copy(s - 2, b).wait()
                transpose_block(b)
                out_copy(s, b).start()
            return carry

        lax.fori_loop(0, SEQ // 2, body, 0)
        for s in (SEQ - 2, SEQ - 1):
            out_copy(s, s % 2).wait()

    return gather_kernel


_GATHER = _make_kernel()


def kernel(indices, table):
    idx_t = indices.T
    out_t = _GATHER(idx_t, table)
    return out_t.transpose(2, 0, 1)


# speedup vs baseline: 1.5233x; 1.5233x over previous
import functools

import jax
import jax.numpy as jnp
from jax import lax
from jax.experimental import pallas as pl
from jax.experimental.pallas import tpu as pltpu
from jax.experimental.pallas import tpu_sc as plsc

BATCH = 16384
SEQ = 200
EMBED = 32
NUM_WORKERS = 32
BPW = BATCH // NUM_WORKERS


def _make_kernel():
    mesh = plsc.VectorSubcoreMesh(core_axis_name="c", subcore_axis_name="s")

    @functools.partial(
        pl.kernel,
        mesh=mesh,
        out_type=jax.ShapeDtypeStruct((SEQ, BATCH, EMBED), jnp.float32),
        scratch_types=[
            pltpu.VMEM((2, BPW), jnp.int32),
            pltpu.VMEM((2, BPW, EMBED), jnp.float32),
            [pltpu.SemaphoreType.DMA] * 2,
            [pltpu.SemaphoreType.DMA] * 2,
            [pltpu.SemaphoreType.DMA] * 2,
        ],
        compiler_params=pltpu.CompilerParams(use_tc_tiling_on_sc=False),
    )
    def gather_kernel(idx_hbm, table_hbm, out_hbm, idx_v, rows_v,
                      idx_sems, out_sems, gsems):
        wid = lax.axis_index("s") * 2 + lax.axis_index("c")
        b0 = pl.multiple_of(wid * BPW, BPW)

        def idx_copy(s, b):
            return pltpu.make_async_copy(
                idx_hbm.at[s, pl.ds(b0, BPW)], idx_v.at[b], idx_sems[b])

        def gather(b):
            return pltpu.make_async_copy(
                table_hbm.at[idx_v.at[b]], rows_v.at[b], gsems[b])

        def out_copy(s, b):
            return pltpu.make_async_copy(
                rows_v.at[b], out_hbm.at[s, pl.ds(b0, BPW)], out_sems[b])

        idx_copy(0, 0).start()
        idx_copy(0, 0).wait()
        gather(0).start()
        idx_copy(1, 1).start()

        def body(j, carry):
            for b in range(2):
                s = j * 2 + b
                bn = 1 - b
                @pl.when(s >= 1)
                def _():
                    out_copy(s - 1, bn).wait()
                @pl.when(s + 1 < SEQ)
                def _():
                    idx_copy(s + 1, bn).wait()
                    gather(bn).start()
                gather(b).wait()
                @pl.when(s + 2 < SEQ)
                def _():
                    idx_copy(s + 2, b).start()
                out_copy(s, b).start()
            return carry

        lax.fori_loop(0, SEQ // 2, body, 0)
        out_copy(SEQ - 1, 1).wait()

    return gather_kernel


_GATHER = _make_kernel()


def kernel(indices, table):
    idx_t = indices.T
    out_t = _GATHER(idx_t, table)
    return out_t.transpose(1, 0, 2)
